# baseline (speedup 1.0000x reference)
import functools
import math

import jax
import jax.numpy as jnp
from jax.experimental import pallas as pl
from jax.experimental.pallas import tpu as pltpu

N = 4096
C = 768
E = 64
K = 2
H = 3072
CAP = 160

TB = 512
HB = 512
NT = N // TB
NH = H // HB


def _gelu(v):
    return 0.5 * v * (1.0 + jax.lax.erf(v * (1.0 / math.sqrt(2.0))))


def _router_body(x_ref, wr_ref, br_ref, eb_ref, idx_ref, w_ref):
    x = x_ref[...]
    z = jnp.dot(x, wr_ref[...], preferred_element_type=jnp.float32)
    z = z + br_ref[0, :][None, :]
    s = jax.nn.sigmoid(z) + eb_ref[0, :][None, :]
    logits = jnp.log(jnp.clip(s, 1e-12, None))
    m = jnp.max(logits, axis=-1, keepdims=True)
    ex = jnp.exp(logits - m)
    probs = ex / jnp.sum(ex, axis=-1, keepdims=True)

    iota = jax.lax.broadcasted_iota(jnp.int32, probs.shape, 1)
    big = jnp.int32(E)
    m1 = jnp.max(probs, axis=-1, keepdims=True)
    i1 = jnp.min(jnp.where(probs == m1, iota, big), axis=-1, keepdims=True)
    masked = jnp.where(iota == i1, -jnp.inf, probs)
    m2 = jnp.max(masked, axis=-1, keepdims=True)
    i2 = jnp.min(jnp.where(masked == m2, iota, big), axis=-1, keepdims=True)

    tot = jnp.clip(m1 + m2, 1e-12, None)
    idx_ref[...] = jnp.concatenate([i1, i2], axis=1)
    w_ref[...] = jnp.concatenate([m1 / tot, m2 / tot], axis=1)


def _router(x, Wr, br, eb):
    return pl.pallas_call(
        _router_body,
        grid=(NT,),
        in_specs=[
            pl.BlockSpec((TB, C), lambda t: (t, 0)),
            pl.BlockSpec((C, E), lambda t: (0, 0)),
            pl.BlockSpec((1, E), lambda t: (0, 0)),
            pl.BlockSpec((1, E), lambda t: (0, 0)),
        ],
        out_specs=[
            pl.BlockSpec((TB, K), lambda t: (t, 0)),
            pl.BlockSpec((TB, K), lambda t: (t, 0)),
        ],
        out_shape=[
            jax.ShapeDtypeStruct((N, K), jnp.int32),
            jax.ShapeDtypeStruct((N, K), jnp.float32),
        ],
    )(x, Wr, br.reshape(1, E), eb.reshape(1, E))


def _expert_body(xe_ref, w1_ref, b1_ref, w2_ref, b2_ref, wt_ref, out_ref):
    hb = pl.program_id(1)

    @pl.when(hb == 0)
    def _():
        out_ref[...] = jnp.broadcast_to(b2_ref[0, 0, :][None, None, :],
                                        out_ref.shape)

    xe = xe_ref[0]
    u = jnp.dot(xe, w1_ref[0], preferred_element_type=jnp.float32)
    h = _gelu(u + b1_ref[0, 0, :][None, :])
    out_ref[...] += jnp.dot(h, w2_ref[0],
                            preferred_element_type=jnp.float32)[None]

    @pl.when(hb == NH - 1)
    def _():
        out_ref[...] = out_ref[...] * wt_ref[0, 0, :][None, :, None]


def _expert_ffn(xe, W1, b1, W2, b2, w_eff):
    return pl.pallas_call(
        _expert_body,
        grid=(E, NH),
        in_specs=[
            pl.BlockSpec((1, CAP, C), lambda e, h: (e, 0, 0)),
            pl.BlockSpec((1, C, HB), lambda e, h: (e, 0, h)),
            pl.BlockSpec((1, 1, HB), lambda e, h: (e, 0, h)),
            pl.BlockSpec((1, HB, C), lambda e, h: (e, h, 0)),
            pl.BlockSpec((1, 1, C), lambda e, h: (e, 0, 0)),
            pl.BlockSpec((1, 1, CAP), lambda e, h: (e, 0, 0)),
        ],
        out_specs=pl.BlockSpec((1, CAP, C), lambda e, h: (e, 0, 0)),
        out_shape=jax.ShapeDtypeStruct((E, CAP, C), jnp.float32),
        compiler_params=pltpu.CompilerParams(
            dimension_semantics=("arbitrary", "arbitrary"),
        ),
    )(xe, W1, b1.reshape(E, 1, H), W2, b2.reshape(E, 1, C),
      w_eff.reshape(E, 1, CAP))


def _shared_body(x_ref, w1_ref, b1_ref, w2_ref, b2_ref, y_ref, out_ref):
    hb = pl.program_id(1)

    @pl.when(hb == 0)
    def _():
        out_ref[...] = y_ref[...] + b2_ref[0, :][None, :]

    u = jnp.dot(x_ref[...], w1_ref[...], preferred_element_type=jnp.float32)
    h = _gelu(u + b1_ref[0, :][None, :])
    out_ref[...] += jnp.dot(h, w2_ref[...], preferred_element_type=jnp.float32)


def _shared_ffn(x, Ws1, bs1, Ws2, bs2, y):
    return pl.pallas_call(
        _shared_body,
        grid=(NT, NH),
        in_specs=[
            pl.BlockSpec((TB, C), lambda t, h: (t, 0)),
            pl.BlockSpec((C, HB), lambda t, h: (0, h)),
            pl.BlockSpec((1, HB), lambda t, h: (0, h)),
            pl.BlockSpec((HB, C), lambda t, h: (h, 0)),
            pl.BlockSpec((1, C), lambda t, h: (0, 0)),
            pl.BlockSpec((TB, C), lambda t, h: (t, 0)),
        ],
        out_specs=pl.BlockSpec((TB, C), lambda t, h: (t, 0)),
        out_shape=jax.ShapeDtypeStruct((N, C), jnp.float32),
        compiler_params=pltpu.CompilerParams(
            dimension_semantics=("arbitrary", "arbitrary"),
        ),
    )(x, Ws1, bs1.reshape(1, H), Ws2, bs2.reshape(1, C), y)


def kernel(x, Wr, br, expert_bias, W1, b1, W2, b2, Ws1, bs1, Ws2, bs2):
    idx, w = _router(x, Wr, br, expert_bias)

    pe = idx.reshape(-1)
    pw = w.reshape(-1)
    pn = jnp.repeat(jnp.arange(N), K)
    order = jnp.argsort(pe.astype(jnp.float32) * 2.0 - pw)
    se = pe[order]
    sw = pw[order]
    sn = pn[order]
    first = jnp.searchsorted(se, se, side='left')
    rank = jnp.arange(N * K) - first
    buf_tok = jnp.full((E, CAP), -1, jnp.int32).at[se, rank].set(
        sn.astype(jnp.int32), mode='drop')
    buf_w = jnp.zeros((E, CAP), x.dtype).at[se, rank].set(sw, mode='drop')
    valid = (buf_tok >= 0)
    tok = jnp.clip(buf_tok, 0, None)
    w_eff = buf_w * valid.astype(x.dtype)

    xe = x[tok]
    oe = _expert_ffn(xe, W1, b1, W2, b2, w_eff)
    y = jnp.zeros((N, C), x.dtype).at[tok.reshape(-1)].add(oe.reshape(-1, C))

    return _shared_ffn(x, Ws1, bs1, Ws2, bs2, y)


# speedup vs baseline: 1.0301x; 1.0301x over previous
import functools
import math

import jax
import jax.numpy as jnp
from jax import lax
from jax.experimental import pallas as pl
from jax.experimental.pallas import tpu as pltpu
from jax.experimental.pallas import tpu_sc as plsc

N = 4096
C = 768
E = 64
K = 2
H = 3072
CAP = 160

TB = 512
HB = 512
NT = N // TB
NH = H // HB


def _gelu(v):
    return 0.5 * v * (1.0 + jax.lax.erf(v * (1.0 / math.sqrt(2.0))))


def _router_body(x_ref, wr_ref, br_ref, eb_ref, idx_ref, w_ref):
    x = x_ref[...]
    z = jnp.dot(x, wr_ref[...], preferred_element_type=jnp.float32)
    z = z + br_ref[0, :][None, :]
    s = jax.nn.sigmoid(z) + eb_ref[0, :][None, :]
    logits = jnp.log(jnp.clip(s, 1e-12, None))
    m = jnp.max(logits, axis=-1, keepdims=True)
    ex = jnp.exp(logits - m)
    probs = ex / jnp.sum(ex, axis=-1, keepdims=True)

    iota = jax.lax.broadcasted_iota(jnp.int32, probs.shape, 1)
    big = jnp.int32(E)
    m1 = jnp.max(probs, axis=-1, keepdims=True)
    i1 = jnp.min(jnp.where(probs == m1, iota, big), axis=-1, keepdims=True)
    masked = jnp.where(iota == i1, -jnp.inf, probs)
    m2 = jnp.max(masked, axis=-1, keepdims=True)
    i2 = jnp.min(jnp.where(masked == m2, iota, big), axis=-1, keepdims=True)

    tot = jnp.clip(m1 + m2, 1e-12, None)
    idx_ref[...] = jnp.concatenate([i1, i2], axis=1)
    w_ref[...] = jnp.concatenate([m1 / tot, m2 / tot], axis=1)


def _router(x, Wr, br, eb):
    return pl.pallas_call(
        _router_body,
        grid=(NT,),
        in_specs=[
            pl.BlockSpec((TB, C), lambda t: (t, 0)),
            pl.BlockSpec((C, E), lambda t: (0, 0)),
            pl.BlockSpec((1, E), lambda t: (0, 0)),
            pl.BlockSpec((1, E), lambda t: (0, 0)),
        ],
        out_specs=[
            pl.BlockSpec((TB, K), lambda t: (t, 0)),
            pl.BlockSpec((TB, K), lambda t: (t, 0)),
        ],
        out_shape=[
            jax.ShapeDtypeStruct((N, K), jnp.int32),
            jax.ShapeDtypeStruct((N, K), jnp.float32),
        ],
    )(x, Wr, br.reshape(1, E), eb.reshape(1, E))


def _expert_body(xe_ref, w1_ref, b1_ref, w2_ref, b2_ref, wt_ref, out_ref):
    hb = pl.program_id(1)

    @pl.when(hb == 0)
    def _():
        out_ref[...] = jnp.broadcast_to(b2_ref[0, 0, :][None, None, :],
                                        out_ref.shape)

    xe = xe_ref[0]
    u = jnp.dot(xe, w1_ref[0], preferred_element_type=jnp.float32)
    h = _gelu(u + b1_ref[0, 0, :][None, :])
    out_ref[...] += jnp.dot(h, w2_ref[0],
                            preferred_element_type=jnp.float32)[None]

    @pl.when(hb == NH - 1)
    def _():
        out_ref[...] = out_ref[...] * wt_ref[0, 0, :][None, :, None]


def _expert_ffn(xe, W1, b1, W2, b2, w_eff_pad):
    ec = lambda e: jnp.minimum(e, E - 1)
    return pl.pallas_call(
        _expert_body,
        grid=(E + 1, NH),
        in_specs=[
            pl.BlockSpec((1, CAP, C), lambda e, h: (ec(e), 0, 0)),
            pl.BlockSpec((1, C, HB), lambda e, h: (ec(e), 0, h)),
            pl.BlockSpec((1, 1, HB), lambda e, h: (ec(e), 0, h)),
            pl.BlockSpec((1, HB, C), lambda e, h: (ec(e), h, 0)),
            pl.BlockSpec((1, 1, C), lambda e, h: (ec(e), 0, 0)),
            pl.BlockSpec((1, 1, CAP), lambda e, h: (e, 0, 0)),
        ],
        out_specs=pl.BlockSpec((1, CAP, C), lambda e, h: (e, 0, 0)),
        out_shape=jax.ShapeDtypeStruct((E + 1, CAP, C), jnp.float32),
        compiler_params=pltpu.CompilerParams(
            dimension_semantics=("arbitrary", "arbitrary"),
        ),
    )(xe, W1, b1.reshape(E, 1, H), W2, b2.reshape(E, 1, C), w_eff_pad)


def _shared_body(x_ref, w1_ref, b1_ref, w2_ref, b2_ref, out_ref):
    hb = pl.program_id(1)

    @pl.when(hb == 0)
    def _():
        out_ref[...] = jnp.broadcast_to(b2_ref[0, :][None, :], out_ref.shape)

    u = jnp.dot(x_ref[...], w1_ref[...], preferred_element_type=jnp.float32)
    h = _gelu(u + b1_ref[0, :][None, :])
    out_ref[...] += jnp.dot(h, w2_ref[...], preferred_element_type=jnp.float32)


def _shared_ffn(x, Ws1, bs1, Ws2, bs2):
    return pl.pallas_call(
        _shared_body,
        grid=(NT, NH),
        in_specs=[
            pl.BlockSpec((TB, C), lambda t, h: (t, 0)),
            pl.BlockSpec((C, HB), lambda t, h: (0, h)),
            pl.BlockSpec((1, HB), lambda t, h: (0, h)),
            pl.BlockSpec((HB, C), lambda t, h: (h, 0)),
            pl.BlockSpec((1, C), lambda t, h: (0, 0)),
        ],
        out_specs=pl.BlockSpec((TB, C), lambda t, h: (t, 0)),
        out_shape=jax.ShapeDtypeStruct((N, C), jnp.float32),
        compiler_params=pltpu.CompilerParams(
            dimension_semantics=("arbitrary", "arbitrary"),
        ),
    )(x, Ws1, bs1.reshape(1, H), Ws2, bs2.reshape(1, C))


_SC_MESH = plsc.VectorSubcoreMesh(core_axis_name="c", subcore_axis_name="s")
_NW = 32
_RPW = (E * CAP) // _NW
_GCH = 4
_GROWS = _RPW // _GCH


def _sc_gather(x, tok_flat):
    @functools.partial(
        pl.kernel,
        mesh=_SC_MESH,
        out_type=jax.ShapeDtypeStruct((E * CAP, C), jnp.float32),
        scratch_types=[
            pltpu.VMEM((_GCH, _GROWS), jnp.int32),
            pltpu.VMEM((_GROWS, C), jnp.float32),
            pltpu.SemaphoreType.DMA,
        ],
    )
    def gather_k(x_hbm, tok_hbm, xe_hbm, idx_v, rows_v, sem):
        wid = lax.axis_index("s") * 2 + lax.axis_index("c")
        base = wid * _RPW
        for ch in range(_GCH):
            pltpu.sync_copy(tok_hbm.at[pl.ds(base + ch * _GROWS, _GROWS)],
                            idx_v.at[ch])
        for ch in range(_GCH):
            pltpu.async_copy(x_hbm.at[idx_v.at[ch]], rows_v, sem).wait()
            pltpu.sync_copy(rows_v,
                            xe_hbm.at[pl.ds(base + ch * _GROWS, _GROWS)])

    return gather_k(x, tok_flat)


_CTOK = 32
_TPW = N // _NW
_CCH = _TPW // _CTOK
_NV = C // 16


def _sc_combine(oew, inv_flat, shared):
    @functools.partial(
        pl.kernel,
        mesh=_SC_MESH,
        out_type=jax.ShapeDtypeStruct((N, C), jnp.float32),
        scratch_types=[
            pltpu.VMEM((2 * _CTOK,), jnp.int32),
            pltpu.VMEM((2 * _CTOK, C), jnp.float32),
            pltpu.VMEM((_CTOK, C), jnp.float32),
            pltpu.SemaphoreType.DMA,
        ],
    )
    def combine_k(oew_hbm, inv_hbm, sh_hbm, out_hbm,
                  inv_v, rows_v, acc_v, sem):
        wid = lax.axis_index("s") * 2 + lax.axis_index("c")
        t0 = wid * _TPW
        for ch in range(_CCH):
            tb = t0 + ch * _CTOK
            pltpu.sync_copy(inv_hbm.at[pl.ds(2 * tb, 2 * _CTOK)], inv_v)
            pltpu.async_copy(oew_hbm.at[inv_v], rows_v, sem).wait()
            pltpu.sync_copy(sh_hbm.at[pl.ds(tb, _CTOK)], acc_v)

            def tok_body(t, _):
                def vreg_body(j, _):
                    s = pl.ds(j * 16, 16)
                    acc_v[t, s] = (acc_v[t, s] + rows_v[2 * t, s]
                                   + rows_v[2 * t + 1, s])
                    return 0

                return lax.fori_loop(0, _NV, vreg_body, 0)

            lax.fori_loop(0, _CTOK, tok_body, 0)
            pltpu.sync_copy(acc_v, out_hbm.at[pl.ds(tb, _CTOK)])

    return combine_k(oew, inv_flat, shared)


def kernel(x, Wr, br, expert_bias, W1, b1, W2, b2, Ws1, bs1, Ws2, bs2):
    idx, w = _router(x, Wr, br, expert_bias)

    pe = idx.reshape(-1)
    pw = w.reshape(-1)
    pn = jnp.repeat(jnp.arange(N), K)
    order = jnp.argsort(pe.astype(jnp.float32) * 2.0 - pw)
    se = pe[order]
    sw = pw[order]
    sn = pn[order]
    first = jnp.searchsorted(se, se, side='left')
    rank = jnp.arange(N * K) - first
    buf_tok = jnp.full((E, CAP), -1, jnp.int32).at[se, rank].set(
        sn.astype(jnp.int32), mode='drop')
    buf_w = jnp.zeros((E, CAP), x.dtype).at[se, rank].set(sw, mode='drop')
    valid = (buf_tok >= 0)
    tok = jnp.clip(buf_tok, 0, None)
    w_eff = buf_w * valid.astype(x.dtype)
    w_eff_pad = jnp.concatenate(
        [w_eff.reshape(E, 1, CAP), jnp.zeros((1, 1, CAP), x.dtype)], axis=0)
    keep = rank < CAP
    inv = jnp.full((N * K,), E * CAP, jnp.int32).at[order].set(
        jnp.where(keep, se * CAP + rank, E * CAP).astype(jnp.int32))

    tok_flat = tok.reshape(-1)
    xe = _sc_gather(x, tok_flat).reshape(E, CAP, C)
    shared = _shared_ffn(x, Ws1, bs1, Ws2, bs2)
    oe = _expert_ffn(xe, W1, b1, W2, b2, w_eff_pad)
    return _sc_combine(oe.reshape((E + 1) * CAP, C), inv, shared)
